# Initial kernel scaffold; baseline (speedup 1.0000x reference)
import jax
import jax.numpy as jnp
from jax.experimental import pallas as pl


def kernel(x, cluster):
    raise NotImplementedError("write your pallas kernel here")


# speedup vs baseline: 1.0471x; 1.0471x over previous
import functools

import jax
import jax.numpy as jnp
from jax import lax
from jax.experimental import pallas as pl
from jax.experimental.pallas import tpu as pltpu
from jax.experimental.pallas import tpu_sc as plsc

N = 320000
D = 128
NUM_SEGMENTS = 80000

NC = 2
NS = 16
NW = NC * NS

SEG_W = 250
WPT = NUM_SEGMENTS // (NW * SEG_W)
NUM_WINDOWS = NUM_SEGMENTS // SEG_W
C = 128
NEG_INF = float("-inf")


def _seg_max_body(x_hbm, cl_hbm, bounds_hbm, out_hbm, x_v, cl_v, out_v, b_v):
    wid = lax.axis_index("s") * NC + lax.axis_index("c")

    def window(w, carry):
        W = wid * WPT + w
        ws = W * SEG_W

        pltpu.sync_copy(bounds_hbm.at[W], b_v)
        rb0 = b_v[...][0]
        pltpu.sync_copy(bounds_hbm.at[W + 1], b_v)
        rb1 = b_v[...][0]

        def init_row(s, c2):
            for j in range(D // 16):
                out_v[s, pl.ds(j * 16, 16)] = jnp.full((16,), NEG_INF, jnp.float32)
            return c2

        lax.fori_loop(0, SEG_W, init_row, 0)

        def chunk(i, c2):
            pltpu.sync_copy(x_hbm.at[pl.ds(i * C, C)], x_v)
            pltpu.sync_copy(cl_hbm.at[pl.ds(i * C, C)], cl_v)

            def group(g, c3):
                cvec = cl_v[pl.ds(g * 16, 16)]
                for k in range(16):
                    r = g * 16 + k
                    c = cvec[k]
                    local = c - ws
                    inw = jnp.logical_and(local >= 0, local < SEG_W)
                    lidx = jnp.clip(local, 0, SEG_W - 1)
                    pen = jnp.where(inw, jnp.float32(0.0), NEG_INF)
                    for j in range(D // 16):
                        v = x_v[r, pl.ds(j * 16, 16)] + pen
                        o = out_v[lidx, pl.ds(j * 16, 16)]
                        out_v[lidx, pl.ds(j * 16, 16)] = jnp.maximum(o, v)
                return c3

            lax.fori_loop(0, C // 16, group, c2)
            return c2

        lax.fori_loop(rb0 // C, (rb1 + C - 1) // C, chunk, 0)

        def fin_row(s, c2):
            for j in range(D // 16):
                v = out_v[s, pl.ds(j * 16, 16)]
                out_v[s, pl.ds(j * 16, 16)] = jnp.where(v == NEG_INF, jnp.float32(0.0), v)
            return c2

        lax.fori_loop(0, SEG_W, fin_row, 0)
        pltpu.sync_copy(out_v, out_hbm.at[pl.ds(ws, SEG_W)])
        return carry

    lax.fori_loop(0, WPT, window, 0)


_seg_max = functools.partial(
    pl.kernel,
    out_type=jax.ShapeDtypeStruct((NUM_SEGMENTS, D), jnp.float32),
    mesh=plsc.VectorSubcoreMesh(core_axis_name="c", subcore_axis_name="s"),
    compiler_params=pltpu.CompilerParams(use_tc_tiling_on_sc=False),
    scratch_types=[
        pltpu.VMEM((C, D), jnp.float32),
        pltpu.VMEM((C,), jnp.int32),
        pltpu.VMEM((SEG_W, D), jnp.float32),
        pltpu.VMEM((16,), jnp.int32),
    ],
)(_seg_max_body)


@jax.jit
def kernel(x, cluster):
    edges = jnp.arange(0, NUM_SEGMENTS + 1, SEG_W, dtype=jnp.int32)
    bounds = jnp.searchsorted(cluster, edges).astype(jnp.int32)
    bounds_b = jnp.broadcast_to(bounds[:, None], (NUM_WINDOWS + 1, 16))
    return _seg_max(x, cluster, bounds_b + 0)


# speedup vs baseline: 2.6315x; 2.5133x over previous
import functools

import jax
import jax.numpy as jnp
from jax import lax
from jax.experimental import pallas as pl
from jax.experimental.pallas import tpu as pltpu
from jax.experimental.pallas import tpu_sc as plsc

N = 320000
D = 128
NUM_SEGMENTS = 80000

NC = 2
NS = 16
NW = NC * NS

SEG_W = 250
WPT = NUM_SEGMENTS // (NW * SEG_W)
NUM_WINDOWS = NUM_SEGMENTS // SEG_W
C = 128
NCH = N // C
NJ = D // 16
NEG_INF = float("-inf")


def _seg_max_body(x_hbm, cl_hbm, bounds_hbm, out_hbm,
                  x_v, cl_v, out_v, b_v, sem_x, sem_cl, sem_out):
    wid = lax.axis_index("s") * NC + lax.axis_index("c")

    def issue(i, s):
        pltpu.async_copy(x_hbm.at[pl.ds(i * C * D, C * D)],
                         x_v.at[pl.ds(s * C * D, C * D)], sem_x)
        pltpu.async_copy(cl_hbm.at[pl.ds(i * C, C)],
                         cl_v.at[pl.ds(s * C, C)], sem_cl)

    def wait_in():
        pltpu.make_async_copy(x_hbm.at[pl.ds(0, C * D)],
                              x_v.at[pl.ds(0, C * D)], sem_x).wait()
        pltpu.make_async_copy(cl_hbm.at[pl.ds(0, C)],
                              cl_v.at[pl.ds(0, C)], sem_cl).wait()

    def window(w, carry):
        W = wid * WPT + w
        ws = W * SEG_W
        ob = w % 2

        pltpu.sync_copy(bounds_hbm.at[W], b_v)
        rb0 = b_v[...][0]
        pltpu.sync_copy(bounds_hbm.at[W + 1], b_v)
        rb1 = b_v[...][0]
        i0 = rb0 // C
        i1 = (rb1 + C - 1) // C

        @pl.when(w >= 2)
        def _():
            pltpu.make_async_copy(
                out_v.at[pl.ds(0, SEG_W * D)], out_hbm.at[pl.ds(0, SEG_W * D)],
                sem_out).wait()

        obase = ob * SEG_W

        def init_row(s, c2):
            for j in range(NJ):
                out_v[pl.ds((obase + s) * D + j * 16, 16)] = jnp.zeros(
                    (16,), jnp.float32)
            return c2

        lax.fori_loop(0, SEG_W, init_row, 0)

        issue(jnp.minimum(i0, NCH - 1), 0)

        def chunk(i, acc_cur):
            s = (i - i0) % 2
            wait_in()
            issue(jnp.minimum(i + 1, i1 - 1), 1 - s)

            def group(g, acc_cur2):
                accs, cur = acc_cur2
                cvec = cl_v[pl.ds(s * C + g * 16, 16)]
                for k in range(16):
                    c = cvec[k]
                    local = c - ws
                    changed = local != cur
                    store = jnp.logical_and(
                        changed,
                        jnp.logical_and(cur >= 0, cur < SEG_W))

                    @pl.when(store)
                    def _(accs=accs, cur=cur):
                        for j in range(NJ):
                            out_v[pl.ds((obase + cur) * D + j * 16, 16)] = accs[j]

                    pen = jnp.where(changed, jnp.float32(NEG_INF),
                                    jnp.float32(0.0))
                    r = g * 16 + k
                    accs = tuple(
                        jnp.maximum(x_v[pl.ds((s * C + r) * D + j * 16, 16)],
                                    accs[j] + pen)
                        for j in range(NJ))
                    cur = local
                return (accs, cur)

            return lax.fori_loop(0, C // 16, group, acc_cur)

        acc0 = tuple(jnp.zeros((16,), jnp.float32) for _ in range(NJ))
        accs, cur = lax.fori_loop(i0, i1, chunk, (acc0, jnp.int32(-1)))
        wait_in()

        @pl.when(jnp.logical_and(cur >= 0, cur < SEG_W))
        def _():
            for j in range(NJ):
                out_v[pl.ds((obase + cur) * D + j * 16, 16)] = accs[j]

        pltpu.async_copy(out_v.at[pl.ds(obase * D, SEG_W * D)],
                         out_hbm.at[pl.ds(ws * D, SEG_W * D)], sem_out)
        return carry

    lax.fori_loop(0, WPT, window, 0)
    pltpu.make_async_copy(out_v.at[pl.ds(0, SEG_W * D)],
                          out_hbm.at[pl.ds(0, SEG_W * D)], sem_out).wait()
    pltpu.make_async_copy(out_v.at[pl.ds(0, SEG_W * D)],
                          out_hbm.at[pl.ds(0, SEG_W * D)], sem_out).wait()


_seg_max = functools.partial(
    pl.kernel,
    out_type=jax.ShapeDtypeStruct((NUM_SEGMENTS * D,), jnp.float32),
    mesh=plsc.VectorSubcoreMesh(core_axis_name="c", subcore_axis_name="s"),
    compiler_params=pltpu.CompilerParams(use_tc_tiling_on_sc=False),
    scratch_types=[
        pltpu.VMEM((2 * C * D,), jnp.float32),
        pltpu.VMEM((2 * C,), jnp.int32),
        pltpu.VMEM((2 * SEG_W * D,), jnp.float32),
        pltpu.VMEM((16,), jnp.int32),
        pltpu.SemaphoreType.DMA,
        pltpu.SemaphoreType.DMA,
        pltpu.SemaphoreType.DMA,
    ],
)(_seg_max_body)


@jax.jit
def kernel(x, cluster):
    edges = jnp.arange(0, NUM_SEGMENTS + 1, SEG_W, dtype=jnp.int32)
    bounds = jnp.searchsorted(cluster, edges).astype(jnp.int32)
    bounds_b = jnp.broadcast_to(bounds[:, None], (NUM_WINDOWS + 1, 16))
    out = _seg_max(x.reshape(-1), cluster, bounds_b + 0)
    return out.reshape(NUM_SEGMENTS, D)


# speedup vs baseline: 2.6335x; 1.0007x over previous
import functools

import jax
import jax.numpy as jnp
from jax import lax
from jax.experimental import pallas as pl
from jax.experimental.pallas import tpu as pltpu
from jax.experimental.pallas import tpu_sc as plsc

N = 320000
D = 128
NUM_SEGMENTS = 80000

NC = 2
NS = 16
NW = NC * NS

SEG_W = 250
WPT = NUM_SEGMENTS // (NW * SEG_W)
NUM_WINDOWS = NUM_SEGMENTS // SEG_W
C = 128
NCH = N // C
NJ = D // 16
NEG_INF = float("-inf")


def _seg_max_body(x_hbm, cl_hbm, bounds_hbm, out_hbm,
                  x_v, cl_v, out_v, b_v, sem_x, sem_cl, sem_out):
    wid = lax.axis_index("s") * NC + lax.axis_index("c")

    def issue(i, s):
        pltpu.async_copy(x_hbm.at[pl.ds(i * C * D, C * D)],
                         x_v.at[pl.ds(s * C * D, C * D)], sem_x)
        pltpu.async_copy(cl_hbm.at[pl.ds(i * C, C)],
                         cl_v.at[pl.ds(s * C, C)], sem_cl)

    def wait_in():
        pltpu.make_async_copy(x_hbm.at[pl.ds(0, C * D)],
                              x_v.at[pl.ds(0, C * D)], sem_x).wait()
        pltpu.make_async_copy(cl_hbm.at[pl.ds(0, C)],
                              cl_v.at[pl.ds(0, C)], sem_cl).wait()

    def window(w, carry):
        W = wid * WPT + w
        ws = W * SEG_W
        ob = w % 2

        pltpu.sync_copy(bounds_hbm.at[W], b_v)
        rb0 = b_v[...][0]
        pltpu.sync_copy(bounds_hbm.at[W + 1], b_v)
        rb1 = b_v[...][0]
        i0 = rb0 // C
        i1 = (rb1 + C - 1) // C

        @pl.when(w >= 2)
        def _():
            pltpu.make_async_copy(
                out_v.at[pl.ds(0, SEG_W * D)], out_hbm.at[pl.ds(0, SEG_W * D)],
                sem_out).wait()

        obase = ob * SEG_W

        def init_row(s, c2):
            for j in range(NJ):
                out_v[pl.ds((obase + s) * D + j * 16, 16)] = jnp.zeros(
                    (16,), jnp.float32)
            return c2

        lax.fori_loop(0, SEG_W, init_row, 0)

        issue(jnp.minimum(i0, NCH - 1), 0)

        def chunk(i, acc_cur):
            s = (i - i0) % 2
            wait_in()
            issue(jnp.minimum(i + 1, i1 - 1), 1 - s)

            def group(g, acc_cur2):
                accs, cur = acc_cur2
                cvec = cl_v[pl.ds(s * C + g * 16, 16)]
                locals_v = cvec - ws
                valid_v = jnp.logical_and(locals_v >= 0, locals_v < SEG_W)
                lidx_v = jnp.where(valid_v, locals_v, 2 * SEG_W - obase)
                for k in range(16):
                    lidx = lidx_v[k]
                    changed = lidx != cur
                    pen = jnp.where(changed, jnp.float32(NEG_INF),
                                    jnp.float32(0.0))
                    r = g * 16 + k
                    accs = tuple(
                        jnp.maximum(x_v[pl.ds((s * C + r) * D + j * 16, 16)],
                                    accs[j] + pen)
                        for j in range(NJ))
                    cur = lidx
                    base = (obase + cur) * D
                    for j in range(NJ):
                        out_v[pl.ds(base + j * 16, 16)] = accs[j]
                return (accs, cur)

            return lax.fori_loop(0, C // 16, group, acc_cur)

        acc0 = tuple(jnp.zeros((16,), jnp.float32) for _ in range(NJ))
        accs, cur = lax.fori_loop(i0, i1, chunk, (acc0, jnp.int32(-1)))
        wait_in()

        pltpu.async_copy(out_v.at[pl.ds(obase * D, SEG_W * D)],
                         out_hbm.at[pl.ds(ws * D, SEG_W * D)], sem_out)
        return carry

    lax.fori_loop(0, WPT, window, 0)
    pltpu.make_async_copy(out_v.at[pl.ds(0, SEG_W * D)],
                          out_hbm.at[pl.ds(0, SEG_W * D)], sem_out).wait()
    pltpu.make_async_copy(out_v.at[pl.ds(0, SEG_W * D)],
                          out_hbm.at[pl.ds(0, SEG_W * D)], sem_out).wait()


_seg_max = functools.partial(
    pl.kernel,
    out_type=jax.ShapeDtypeStruct((NUM_SEGMENTS * D,), jnp.float32),
    mesh=plsc.VectorSubcoreMesh(core_axis_name="c", subcore_axis_name="s"),
    compiler_params=pltpu.CompilerParams(use_tc_tiling_on_sc=False),
    scratch_types=[
        pltpu.VMEM((2 * C * D,), jnp.float32),
        pltpu.VMEM((2 * C,), jnp.int32),
        pltpu.VMEM(((2 * SEG_W + 1) * D,), jnp.float32),
        pltpu.VMEM((16,), jnp.int32),
        pltpu.SemaphoreType.DMA,
        pltpu.SemaphoreType.DMA,
        pltpu.SemaphoreType.DMA,
    ],
)(_seg_max_body)


@jax.jit
def kernel(x, cluster):
    edges = jnp.arange(0, NUM_SEGMENTS + 1, SEG_W, dtype=jnp.int32)
    bounds = jnp.searchsorted(cluster, edges).astype(jnp.int32)
    bounds_b = jnp.broadcast_to(bounds[:, None], (NUM_WINDOWS + 1, 16))
    out = _seg_max(x.reshape(-1), cluster, bounds_b + 0)
    return out.reshape(NUM_SEGMENTS, D)
